# baseline (speedup 1.0000x reference)
import jax
import jax.numpy as jnp
from jax import lax
from jax.experimental import pallas as pl
from jax.experimental.pallas import tpu as pltpu
from jax.experimental.pallas import tpu_sc as plsc

_B = 16384
_H = 200
_D = 16
_V = 1000000
_NC = 2
_NS = 16
_NW = _NC * _NS

_LV = 2048
_CTOT = 487
_NKV = 16
_VTAIL = 999424

_BW = _B // _NW
_LB = 20
_NSL = _H // _LB


def _t_body(tabT_hbm, tailT_hbm, w_hbm, t_hbm, wv, tab_a, tab_b, tail_v,
            tv_a, tv_b, sem_a, sem_b, sem_w):
    wid = lax.axis_index("s") * _NC + lax.axis_index("c")
    pltpu.sync_copy(w_hbm, wv)
    wvec = wv[...]
    ws = [wvec[d] for d in range(_D)]

    def v0_of(k):
        return jnp.minimum(wid + 32 * k, _CTOT) * _LV

    def start(k, buf, sem):
        pltpu.async_copy(tabT_hbm.at[:, pl.ds(v0_of(k), _LV)], buf, sem)

    def process(buf, tvbuf, k):
        def group(j, carry):
            for u in range(2):
                o = (j * 2 + u) * 16
                m = [buf[d, pl.ds(o, 16)] * ws[d] for d in range(_D)]
                while len(m) > 1:
                    m = [m[i] + m[i + 1] for i in range(0, len(m), 2)]
                tvbuf[pl.ds(o, 16)] = m[0]
            return carry

        lax.fori_loop(0, _LV // 32, group, 0)
        pltpu.async_copy(tvbuf, t_hbm.at[pl.ds(v0_of(k), _LV)], sem_w)

    def wait_one_write():
        pltpu.make_async_copy(tv_a, t_hbm.at[pl.ds(0, _LV)], sem_w).wait()

    start(0, tab_a, sem_a)
    start(1, tab_b, sem_b)

    def pair(k2, carry):
        ka = 2 * k2
        pltpu.make_async_copy(tabT_hbm.at[:, pl.ds(0, _LV)], tab_a,
                              sem_a).wait()

        @pl.when(ka >= 2)
        def _():
            wait_one_write()

        process(tab_a, tv_a, ka)

        @pl.when(ka + 2 < _NKV)
        def _():
            start(ka + 2, tab_a, sem_a)

        pltpu.make_async_copy(tabT_hbm.at[:, pl.ds(0, _LV)], tab_b,
                              sem_b).wait()

        @pl.when(ka >= 2)
        def _():
            wait_one_write()

        process(tab_b, tv_b, ka + 1)

        @pl.when(ka + 3 < _NKV)
        def _():
            start(ka + 3, tab_b, sem_b)

        return carry

    lax.fori_loop(0, _NKV // 2, pair, 0)
    wait_one_write()
    wait_one_write()

    @pl.when(wid == 0)
    def _():
        pltpu.sync_copy(tailT_hbm, tail_v)

        def tgroup(j, carry):
            o = j * 16
            m = [tail_v[d, pl.ds(o, 16)] * ws[d] for d in range(_D)]
            while len(m) > 1:
                m = [m[i] + m[i + 1] for i in range(0, len(m), 2)]
            tv_a[pl.ds(o, 16)] = m[0]
            return carry

        lax.fori_loop(0, 576 // 16, tgroup, 0)
        pltpu.sync_copy(tv_a.at[pl.ds(0, 576)], t_hbm.at[pl.ds(_VTAIL, 576)])


_TSH = 62528


def _gather_body(xf_hbm, t_hbm, s_hbm, idx_a, idx_b, idx_c, vals_a, vals_b,
                 vals_c, acc_v, tsh, semi_a, semi_b, semg):
    wid = lax.axis_index("s") * _NC + lax.axis_index("c")
    b0 = wid * _BW

    idx_bufs = [idx_a, idx_b, idx_c]
    val_bufs = [vals_a, vals_b, vals_c]

    def fetch_slab(si):
        buf = idx_bufs[si % 3]
        return [pltpu.async_copy(
                    xf_hbm.at[si * _LB + lr, pl.ds(b0, _BW)],
                    buf.at[pl.ds(lr * _BW, _BW)], semi_a)
                for lr in range(_LB)]

    def gather(si):
        return pltpu.async_copy(tsh.at[idx_bufs[si % 3]], val_bufs[si % 3],
                                semg)

    h0 = fetch_slab(0)

    def zero(q, carry):
        acc_v[pl.ds(q * 16, 16)] = jnp.zeros((16,), jnp.float32)
        return carry

    lax.fori_loop(0, _BW // 16, zero, 0)

    sid = lax.axis_index("s")
    st = jnp.minimum(sid * _TSH, _V - _TSH)
    sl = _TSH // 8
    hr = pltpu.async_copy(t_hbm.at[pl.ds(st, sl)],
                          vals_a.at[pl.ds(0, sl)], semg)
    for h in range(8):
        hr.wait()
        if h + 1 < 8:
            hr = pltpu.async_copy(
                t_hbm.at[pl.ds(st + (h + 1) * sl, sl)],
                val_bufs[(h + 1) % 2].at[pl.ds(0, sl)], semg)
        pltpu.sync_copy(val_bufs[h % 2].at[pl.ds(0, sl)],
                        tsh.at[pl.ds(st + h * sl, sl)])
    plsc.subcore_barrier()

    for h in h0:
        h.wait()
    gh = {0: gather(0)}
    pending = fetch_slab(1)
    for h in pending:
        h.wait()
    gh[1] = gather(1)
    pending = fetch_slab(2)
    for si in range(_NSL):
        vals_v = val_bufs[si % 3]
        if si + 2 < _NSL:
            for h in pending:
                h.wait()
            gh[si + 2] = gather(si + 2)
        gh[si].wait()
        if si + 3 < _NSL:
            pending = fetch_slab(si + 3)

        def red(q, carry):
            o = q * 16
            m = [vals_v[pl.ds(lr * _BW + o, 16)] for lr in range(_LB)]
            while len(m) > 1:
                m = [m[i] + m[i + 1] for i in range(0, len(m) - 1, 2)] + \
                    ([m[-1]] if len(m) % 2 else [])
            acc_v[pl.ds(o, 16)] = acc_v[pl.ds(o, 16)] + m[0]
            return carry

        lax.fori_loop(0, _BW // 16, red, 0)

    pltpu.sync_copy(acc_v, s_hbm.at[pl.ds(b0, _BW)])


def _final_body(b_ref, s_ref, out_ref):
    z = s_ref[...] * (1.0 / _H) + b_ref[0]
    y = 1.0 / (1.0 + jnp.exp(-z))
    out_ref[...] = jnp.round(y * 10000.0) / 10000.0


def kernel(x, table, W, b):
    t = pl.kernel(
        _t_body,
        out_type=jax.ShapeDtypeStruct((_V,), jnp.float32),
        mesh=plsc.VectorSubcoreMesh(core_axis_name="c", subcore_axis_name="s"),
        compiler_params=pltpu.CompilerParams(
            needs_layout_passes=False, use_tc_tiling_on_sc=True),
        scratch_types=[
            pltpu.VMEM((_D,), jnp.float32),
            pltpu.VMEM((_D, _LV), jnp.float32),
            pltpu.VMEM((_D, _LV), jnp.float32),
            pltpu.VMEM((_D, _V - _VTAIL), jnp.float32),
            pltpu.VMEM((_LV,), jnp.float32),
            pltpu.VMEM((_LV,), jnp.float32),
            pltpu.SemaphoreType.DMA,
            pltpu.SemaphoreType.DMA,
            pltpu.SemaphoreType.DMA,
        ],
    )(table.T, table[_VTAIL:, :].T, W.reshape(_D))

    s = pl.kernel(
        _gather_body,
        out_type=jax.ShapeDtypeStruct((_B,), jnp.float32),
        mesh=plsc.VectorSubcoreMesh(core_axis_name="c", subcore_axis_name="s"),
        compiler_params=pltpu.CompilerParams(
            needs_layout_passes=False, use_tc_tiling_on_sc=True),
        scratch_types=[
            pltpu.VMEM((_LB * _BW,), jnp.int32),
            pltpu.VMEM((_LB * _BW,), jnp.int32),
            pltpu.VMEM((_LB * _BW,), jnp.int32),
            pltpu.VMEM((_LB * _BW,), jnp.float32),
            pltpu.VMEM((_LB * _BW,), jnp.float32),
            pltpu.VMEM((_LB * _BW,), jnp.float32),
            pltpu.VMEM((_BW,), jnp.float32),
            pltpu.VMEM_SHARED((_V,), jnp.float32),
            pltpu.SemaphoreType.DMA,
            pltpu.SemaphoreType.DMA,
            pltpu.SemaphoreType.DMA,
        ],
    )(x.T, t)

    out = pl.pallas_call(
        _final_body,
        grid=(1,),
        in_specs=[
            pl.BlockSpec(memory_space=pltpu.SMEM),
            pl.BlockSpec((128, 128), lambda i: (0, 0)),
        ],
        out_specs=pl.BlockSpec((128, 128), lambda i: (0, 0)),
        out_shape=jax.ShapeDtypeStruct((128, 128), jnp.float32),
    )(b, s.reshape(128, 128))
    return out.reshape(_B, 1)


# speedup vs baseline: 1.0002x; 1.0002x over previous
import jax
import jax.numpy as jnp
from jax import lax
from jax.experimental import pallas as pl
from jax.experimental.pallas import tpu as pltpu
from jax.experimental.pallas import tpu_sc as plsc

_B = 16384
_H = 200
_D = 16
_V = 1000000
_NC = 2
_NS = 16
_NW = _NC * _NS

_LV = 2048
_CTOT = 487
_NKV = 16
_VTAIL = 999424

_BW = _B // _NW
_LB = 20
_NSL = _H // _LB


def _t_body(tabT_hbm, tailT_hbm, w_hbm, t_hbm, wv, tab_a, tab_b, tail_v,
            tv_a, tv_b, sem_a, sem_b, sem_w):
    wid = lax.axis_index("s") * _NC + lax.axis_index("c")
    pltpu.sync_copy(w_hbm, wv)
    wvec = wv[...]
    ws = [wvec[d] for d in range(_D)]

    def v0_of(k):
        return jnp.minimum(wid + 32 * k, _CTOT) * _LV

    def start(k, buf, sem):
        pltpu.async_copy(tabT_hbm.at[:, pl.ds(v0_of(k), _LV)], buf, sem)

    def process(buf, tvbuf, k):
        def group(j, carry):
            for u in range(2):
                o = (j * 2 + u) * 16
                m = [buf[d, pl.ds(o, 16)] * ws[d] for d in range(_D)]
                while len(m) > 1:
                    m = [m[i] + m[i + 1] for i in range(0, len(m), 2)]
                tvbuf[pl.ds(o, 16)] = m[0]
            return carry

        lax.fori_loop(0, _LV // 32, group, 0)
        pltpu.async_copy(tvbuf, t_hbm.at[pl.ds(v0_of(k), _LV)], sem_w)

    def wait_one_write():
        pltpu.make_async_copy(tv_a, t_hbm.at[pl.ds(0, _LV)], sem_w).wait()

    start(0, tab_a, sem_a)
    start(1, tab_b, sem_b)

    def pair(k2, carry):
        ka = 2 * k2
        pltpu.make_async_copy(tabT_hbm.at[:, pl.ds(0, _LV)], tab_a,
                              sem_a).wait()

        @pl.when(ka >= 2)
        def _():
            wait_one_write()

        process(tab_a, tv_a, ka)

        @pl.when(ka + 2 < _NKV)
        def _():
            start(ka + 2, tab_a, sem_a)

        pltpu.make_async_copy(tabT_hbm.at[:, pl.ds(0, _LV)], tab_b,
                              sem_b).wait()

        @pl.when(ka >= 2)
        def _():
            wait_one_write()

        process(tab_b, tv_b, ka + 1)

        @pl.when(ka + 3 < _NKV)
        def _():
            start(ka + 3, tab_b, sem_b)

        return carry

    lax.fori_loop(0, _NKV // 2, pair, 0)
    wait_one_write()
    wait_one_write()

    @pl.when(wid == 0)
    def _():
        pltpu.sync_copy(tailT_hbm, tail_v)

        def tgroup(j, carry):
            o = j * 16
            m = [tail_v[d, pl.ds(o, 16)] * ws[d] for d in range(_D)]
            while len(m) > 1:
                m = [m[i] + m[i + 1] for i in range(0, len(m), 2)]
            tv_a[pl.ds(o, 16)] = m[0]
            return carry

        lax.fori_loop(0, 576 // 16, tgroup, 0)
        pltpu.sync_copy(tv_a.at[pl.ds(0, 576)], t_hbm.at[pl.ds(_VTAIL, 576)])


_TSH = 62528


def _gather_body(xf_hbm, t_hbm, s_hbm, idx_a, idx_b, vals_a, vals_b, acc_v,
                 tsh, semi_a, semi_b, semg):
    wid = lax.axis_index("s") * _NC + lax.axis_index("c")
    b0 = wid * _BW

    idx_bufs = [(idx_a, semi_a), (idx_b, semi_b)]
    val_bufs = [vals_a, vals_b]

    def fetch_slab(si):
        buf, sem = idx_bufs[si % 2]
        return [pltpu.async_copy(
                    xf_hbm.at[si * _LB + lr, pl.ds(b0, _BW)],
                    buf.at[pl.ds(lr * _BW, _BW)], sem)
                for lr in range(_LB)]

    h0 = fetch_slab(0)

    def zero(q, carry):
        acc_v[pl.ds(q * 16, 16)] = jnp.zeros((16,), jnp.float32)
        return carry

    lax.fori_loop(0, _BW // 16, zero, 0)

    sid = lax.axis_index("s")
    st = jnp.minimum(sid * _TSH, _V - _TSH)
    sl = _TSH // 8
    hr = pltpu.async_copy(t_hbm.at[pl.ds(st, sl)],
                          vals_a.at[pl.ds(0, sl)], semg)
    for h in range(8):
        hr.wait()
        if h + 1 < 8:
            hr = pltpu.async_copy(
                t_hbm.at[pl.ds(st + (h + 1) * sl, sl)],
                val_bufs[(h + 1) % 2].at[pl.ds(0, sl)], semg)
        pltpu.sync_copy(val_bufs[h % 2].at[pl.ds(0, sl)],
                        tsh.at[pl.ds(st + h * sl, sl)])
    plsc.subcore_barrier()

    for h in h0:
        h.wait()
    gh = {0: pltpu.async_copy(tsh.at[idx_a], vals_a, semg)}
    pending = fetch_slab(1)
    for si in range(_NSL):
        vals_v = val_bufs[si % 2]
        if si + 1 < _NSL:
            for h in pending:
                h.wait()
            gh[si + 1] = pltpu.async_copy(
                tsh.at[idx_bufs[(si + 1) % 2][0]], val_bufs[(si + 1) % 2],
                semg)
        gh[si].wait()
        if si + 2 < _NSL:
            pending = fetch_slab(si + 2)

        def red(q, carry):
            o = q * 16
            m = [vals_v[pl.ds(lr * _BW + o, 16)] for lr in range(_LB)]
            while len(m) > 1:
                m = [m[i] + m[i + 1] for i in range(0, len(m) - 1, 2)] + \
                    ([m[-1]] if len(m) % 2 else [])
            acc_v[pl.ds(o, 16)] = acc_v[pl.ds(o, 16)] + m[0]
            return carry

        lax.fori_loop(0, _BW // 16, red, 0)

    pltpu.sync_copy(acc_v, s_hbm.at[pl.ds(b0, _BW)])


def _final_body(b_ref, s_ref, out_ref):
    z = s_ref[...] * (1.0 / _H) + b_ref[0]
    y = 1.0 / (1.0 + jnp.exp(-z))
    out_ref[...] = jnp.round(y * 10000.0) / 10000.0


def kernel(x, table, W, b):
    t = pl.kernel(
        _t_body,
        out_type=jax.ShapeDtypeStruct((_V,), jnp.float32),
        mesh=plsc.VectorSubcoreMesh(core_axis_name="c", subcore_axis_name="s"),
        compiler_params=pltpu.CompilerParams(
            needs_layout_passes=False, use_tc_tiling_on_sc=True),
        scratch_types=[
            pltpu.VMEM((_D,), jnp.float32),
            pltpu.VMEM((_D, _LV), jnp.float32),
            pltpu.VMEM((_D, _LV), jnp.float32),
            pltpu.VMEM((_D, _V - _VTAIL), jnp.float32),
            pltpu.VMEM((_LV,), jnp.float32),
            pltpu.VMEM((_LV,), jnp.float32),
            pltpu.SemaphoreType.DMA,
            pltpu.SemaphoreType.DMA,
            pltpu.SemaphoreType.DMA,
        ],
    )(table.T, table[_VTAIL:, :].T, W.reshape(_D))

    s = pl.kernel(
        _gather_body,
        out_type=jax.ShapeDtypeStruct((_B,), jnp.float32),
        mesh=plsc.VectorSubcoreMesh(core_axis_name="c", subcore_axis_name="s"),
        compiler_params=pltpu.CompilerParams(
            needs_layout_passes=False, use_tc_tiling_on_sc=True),
        scratch_types=[
            pltpu.VMEM((_LB * _BW,), jnp.int32),
            pltpu.VMEM((_LB * _BW,), jnp.int32),
            pltpu.VMEM((_LB * _BW,), jnp.float32),
            pltpu.VMEM((_LB * _BW,), jnp.float32),
            pltpu.VMEM((_BW,), jnp.float32),
            pltpu.VMEM_SHARED((_V,), jnp.float32),
            pltpu.SemaphoreType.DMA,
            pltpu.SemaphoreType.DMA,
            pltpu.SemaphoreType.DMA,
        ],
    )(x.T, t)

    out = pl.pallas_call(
        _final_body,
        grid=(1,),
        in_specs=[
            pl.BlockSpec(memory_space=pltpu.SMEM),
            pl.BlockSpec((128, 128), lambda i: (0, 0)),
        ],
        out_specs=pl.BlockSpec((128, 128), lambda i: (0, 0)),
        out_shape=jax.ShapeDtypeStruct((128, 128), jnp.float32),
    )(b, s.reshape(128, 128))
    return out.reshape(_B, 1)
